# baseline (speedup 1.0000x reference)
import functools

import numpy as np
import jax
import jax.numpy as jnp
from jax import lax
from jax.experimental import pallas as pl
from jax.experimental.pallas import tpu as pltpu
from jax.experimental.pallas import tpu_sc as plsc

_P = 8
_V = 1000
_D = 128
_B = 1024
_L = 50
_T = _B * _L
_EPS = 0.01


def _threefry2x32_np(k1, k2, x0, x1):
    def rotl(x, d):
        return ((x << np.uint32(d)) | (x >> np.uint32(32 - d))).astype(np.uint32)

    def rnds(x0, x1, rots):
        for r in rots:
            x0 = (x0 + x1).astype(np.uint32)
            x1 = rotl(x1, r)
            x1 = x1 ^ x0
        return x0, x1

    r0, r1 = (13, 15, 26, 6), (17, 29, 16, 24)
    ks0, ks1 = np.uint32(k1), np.uint32(k2)
    ks2 = np.uint32(ks0 ^ ks1 ^ np.uint32(0x1BD11BDA))
    x0 = (x0 + ks0).astype(np.uint32)
    x1 = (x1 + ks1).astype(np.uint32)
    for i, (ka, kb, rr) in enumerate(
        [(ks1, ks2, r0), (ks2, ks0, r1), (ks0, ks1, r0), (ks1, ks2, r1), (ks2, ks0, r0)]
    ):
        x0, x1 = rnds(x0, x1, rr)
        x0 = (x0 + ka).astype(np.uint32)
        x1 = (x1 + kb + np.uint32(i + 1)).astype(np.uint32)
    return x0, x1


def _emu_np():
    b1, b2 = _threefry2x32_np(
        np.uint32(0), np.uint32(42),
        np.array([0, 0], np.uint32), np.array([0, 1], np.uint32),
    )
    n = _P * _V * _D
    i = np.arange(n, dtype=np.uint64)
    hi = (i >> np.uint64(32)).astype(np.uint32)
    lo = (i & np.uint64(0xFFFFFFFF)).astype(np.uint32)
    bb1, bb2 = _threefry2x32_np(b1[1], b2[1], hi, lo)
    mu = ((bb1 ^ bb2) & np.uint32(1)).astype(np.float32) * 2.0 - 1.0
    return (np.float32(_EPS) * mu).reshape(_P * _V, _D)


_EMU = _emu_np()

_info = plsc.get_sparse_core_info()
_NC = _info.num_cores
_NS = _info.num_subcores
_NW = _NC * _NS
_QP = _NW // _P
_RPT = (_P * _T) // _NW
_C = 64
_NCH = _RPT // _C
_NBUF = 4
_WTAIL_BASE = 896
_WTAIL = _V - _WTAIL_BASE

_mesh = plsc.VectorSubcoreMesh(core_axis_name="c", subcore_axis_name="s")


@functools.partial(
    pl.kernel,
    mesh=_mesh,
    out_type=jax.ShapeDtypeStruct((_P * _T, _D), jnp.float32),
    scratch_types=[
        pltpu.VMEM((_RPT,), jnp.int32),
        pltpu.VMEM((_NBUF, _C, _D), jnp.float32),
        pltpu.VMEM((128, _D), jnp.float32),
        pltpu.VMEM((128,), jnp.int32),
        pltpu.VMEM((_WTAIL,), jnp.int32),
        pltpu.VMEM_SHARED((_P * _V, _D), jnp.float32),
        pltpu.SemaphoreType.DMA((_NBUF,)),
        pltpu.SemaphoreType.DMA((_NBUF,)),
        pltpu.SemaphoreType.DMA,
    ],
)
def _gather(w_hbm, emu_hbm, ids_hbm, out_hbm,
            ids_v, rows_v, wbuf, idxf, idxt, tbl_sh, gsem, wsem, ssem):
    sub = lax.axis_index("s")
    wid = sub * _NC + lax.axis_index("c")
    rbase = wid * _RPT

    @pl.when(sub < _P)
    def _stage():
        v0 = sub * _V
        pltpu.async_copy(emu_hbm.at[pl.ds(v0, _V)], tbl_sh.at[pl.ds(v0, _V)], ssem)

    pltpu.sync_copy(ids_hbm.at[pl.ds((wid % _QP) * _RPT, _RPT)], ids_v)
    poff = (wid // _QP) * _V

    def addoff(i, carry):
        for jj in range(4):
            sl = pl.ds(i * 64 + jj * 16, 16)
            ids_v[sl] = ids_v[sl] + poff
        return carry

    lax.fori_loop(0, _RPT // 64, addoff, 0)

    @pl.when(sub < _P)
    def _stage_wait():
        v0 = sub * _V
        pltpu.make_async_copy(
            emu_hbm.at[pl.ds(v0, _V)], tbl_sh.at[pl.ds(v0, _V)], ssem
        ).wait()

    plsc.subcore_barrier()

    @pl.when(sub < 7)
    def _wadd_full():
        wb = sub * 128
        pltpu.sync_copy(w_hbm.at[pl.ds(wb, 128)], wbuf)
        for jj in range(8):
            idxf[pl.ds(jj * 16, 16)] = lax.iota(jnp.int32, 16) + (wb + jj * 16)
        for p in range(_P):
            pltpu.sync_copy(wbuf, tbl_sh.at[idxf], add=True)
            if p < _P - 1:
                for jj in range(8):
                    sl = pl.ds(jj * 16, 16)
                    idxf[sl] = idxf[sl] + _V

    @pl.when(sub == 7)
    def _wadd_tail():
        wv = wbuf.at[pl.ds(0, _WTAIL)]
        pltpu.sync_copy(w_hbm.at[pl.ds(_WTAIL_BASE, _WTAIL)], wv)
        for p in range(_P):
            for jj in range(_WTAIL // 16):
                idxt[pl.ds(jj * 16, 16)] = lax.iota(jnp.int32, 16) + (
                    p * _V + _WTAIL_BASE + jj * 16
                )
            idxt[pl.ds(_WTAIL - 16, 16)] = lax.iota(jnp.int32, 16) + (
                p * _V + _V - 16
            )
            pltpu.sync_copy(wv, tbl_sh.at[idxt], add=True)

    plsc.subcore_barrier()

    def start_gather(c, b):
        return pltpu.async_copy(
            tbl_sh.at[ids_v.at[pl.ds(c * _C, _C)]], rows_v.at[b], gsem.at[b]
        )

    def wait_gather(c, b):
        pltpu.make_async_copy(
            tbl_sh.at[ids_v.at[pl.ds(c * _C, _C)]], rows_v.at[b], gsem.at[b]
        ).wait()

    def start_write(c, b):
        return pltpu.async_copy(
            rows_v.at[b], out_hbm.at[pl.ds(rbase + c * _C, _C)], wsem.at[b]
        )

    def wait_write(c, b):
        pltpu.make_async_copy(
            rows_v.at[b], out_hbm.at[pl.ds(rbase + c * _C, _C)], wsem.at[b]
        ).wait()

    for b in range(_NBUF):
        start_gather(b, b)
    for c in range(_NBUF - 1):
        wait_gather(c, c)
        start_write(c, c)

    def body(i, carry):
        g = i * _NBUF
        for b in range(_NBUF):
            c = g + b
            wait_write(c - _NBUF, b)
            start_gather(c, b)
            b2 = (b - 1) % _NBUF
            wait_gather(c - 1, b2)
            start_write(c - 1, b2)
        return carry

    lax.fori_loop(1, _NCH // _NBUF, body, 0)

    cB = _NCH - 1
    wait_gather(cB, cB % _NBUF)
    start_write(cB, cB % _NBUF)
    for c in range(_NCH - _NBUF, _NCH):
        wait_write(c, c % _NBUF)


def kernel(input_ids, weight):
    ids_t = input_ids.astype(jnp.int32).T.reshape(_T)
    out = _gather(weight, jnp.asarray(_EMU), ids_t)
    return out.reshape(_P, _L, _B, _D).transpose(0, 2, 1, 3)


# speedup vs baseline: 1.0351x; 1.0351x over previous
import functools

import numpy as np
import jax
import jax.numpy as jnp
from jax import lax
from jax.experimental import pallas as pl
from jax.experimental.pallas import tpu as pltpu
from jax.experimental.pallas import tpu_sc as plsc

_P = 8
_V = 1000
_D = 128
_B = 1024
_L = 50
_T = _B * _L
_EPS = 0.01


def _threefry2x32_np(k1, k2, x0, x1):
    def rotl(x, d):
        return ((x << np.uint32(d)) | (x >> np.uint32(32 - d))).astype(np.uint32)

    def rnds(x0, x1, rots):
        for r in rots:
            x0 = (x0 + x1).astype(np.uint32)
            x1 = rotl(x1, r)
            x1 = x1 ^ x0
        return x0, x1

    r0, r1 = (13, 15, 26, 6), (17, 29, 16, 24)
    ks0, ks1 = np.uint32(k1), np.uint32(k2)
    ks2 = np.uint32(ks0 ^ ks1 ^ np.uint32(0x1BD11BDA))
    x0 = (x0 + ks0).astype(np.uint32)
    x1 = (x1 + ks1).astype(np.uint32)
    for i, (ka, kb, rr) in enumerate(
        [(ks1, ks2, r0), (ks2, ks0, r1), (ks0, ks1, r0), (ks1, ks2, r1), (ks2, ks0, r0)]
    ):
        x0, x1 = rnds(x0, x1, rr)
        x0 = (x0 + ka).astype(np.uint32)
        x1 = (x1 + kb + np.uint32(i + 1)).astype(np.uint32)
    return x0, x1


def _emu_np():
    b1, b2 = _threefry2x32_np(
        np.uint32(0), np.uint32(42),
        np.array([0, 0], np.uint32), np.array([0, 1], np.uint32),
    )
    n = _P * _V * _D
    i = np.arange(n, dtype=np.uint64)
    hi = (i >> np.uint64(32)).astype(np.uint32)
    lo = (i & np.uint64(0xFFFFFFFF)).astype(np.uint32)
    bb1, bb2 = _threefry2x32_np(b1[1], b2[1], hi, lo)
    mu = ((bb1 ^ bb2) & np.uint32(1)).astype(np.float32) * 2.0 - 1.0
    return (np.float32(_EPS) * mu).reshape(_P * _V, _D)


_EMU = _emu_np()

def _prep_body(w_ref, emu_ref, tbl_ref):
    tbl_ref[...] = w_ref[...] + emu_ref[...]


def _prep(weight, emu):
    return pl.pallas_call(
        _prep_body,
        grid=(_P,),
        in_specs=[
            pl.BlockSpec((_V, _D), lambda p: (0, 0)),
            pl.BlockSpec((_V, _D), lambda p: (p, 0)),
        ],
        out_specs=pl.BlockSpec((_V, _D), lambda p: (p, 0)),
        out_shape=jax.ShapeDtypeStruct((_P * _V, _D), jnp.float32),
    )(weight, emu)


_info = plsc.get_sparse_core_info()
_NC = _info.num_cores
_NS = _info.num_subcores
_NW = _NC * _NS
_QP = _NW // _P
_RPT = (_P * _T) // _NW
_C = 80
_NCH = _RPT // _C
_NBUF = 4

_mesh = plsc.VectorSubcoreMesh(core_axis_name="c", subcore_axis_name="s")


@functools.partial(
    pl.kernel,
    mesh=_mesh,
    out_type=jax.ShapeDtypeStruct((_P * _T, _D), jnp.float32),
    scratch_types=[
        pltpu.VMEM((_RPT,), jnp.int32),
        pltpu.VMEM((_NBUF, _C, _D), jnp.float32),
        pltpu.VMEM_SHARED((_P * _V, _D), jnp.float32),
        pltpu.SemaphoreType.DMA((_NBUF,)),
        pltpu.SemaphoreType.DMA((_NBUF,)),
        pltpu.SemaphoreType.DMA,
    ],
)
def _gather(tbl_hbm, ids_hbm, out_hbm, ids_v, rows_v, tbl_sh, gsem, wsem, ssem):
    sub = lax.axis_index("s")
    wid = sub * _NC + lax.axis_index("c")
    rbase = wid * _RPT

    @pl.when(sub < _P)
    def _stage():
        v0 = sub * _V
        pltpu.async_copy(tbl_hbm.at[pl.ds(v0, _V)], tbl_sh.at[pl.ds(v0, _V)], ssem)

    pltpu.sync_copy(ids_hbm.at[pl.ds((wid % _QP) * _RPT, _RPT)], ids_v)
    poff = (wid // _QP) * _V

    def addoff(i, carry):
        for jj in range(4):
            sl = pl.ds(i * 64 + jj * 16, 16)
            ids_v[sl] = ids_v[sl] + poff
        return carry

    lax.fori_loop(0, _RPT // 64, addoff, 0)

    @pl.when(sub < _P)
    def _stage_wait():
        v0 = sub * _V
        pltpu.make_async_copy(
            tbl_hbm.at[pl.ds(v0, _V)], tbl_sh.at[pl.ds(v0, _V)], ssem
        ).wait()

    plsc.subcore_barrier()

    def start_gather(c, b):
        return pltpu.async_copy(
            tbl_sh.at[ids_v.at[pl.ds(c * _C, _C)]], rows_v.at[b], gsem.at[b]
        )

    def wait_gather(c, b):
        pltpu.make_async_copy(
            tbl_sh.at[ids_v.at[pl.ds(c * _C, _C)]], rows_v.at[b], gsem.at[b]
        ).wait()

    def start_write(c, b):
        return pltpu.async_copy(
            rows_v.at[b], out_hbm.at[pl.ds(rbase + c * _C, _C)], wsem.at[b]
        )

    def wait_write(c, b):
        pltpu.make_async_copy(
            rows_v.at[b], out_hbm.at[pl.ds(rbase + c * _C, _C)], wsem.at[b]
        ).wait()

    for b in range(_NBUF):
        start_gather(b, b)
    for c in range(_NBUF - 1):
        wait_gather(c, c)
        start_write(c, c)

    def body(i, carry):
        g = i * _NBUF
        for b in range(_NBUF):
            c = g + b
            wait_write(c - _NBUF, b)
            start_gather(c, b)
            b2 = (b - 1) % _NBUF
            wait_gather(c - 1, b2)
            start_write(c - 1, b2)
        return carry

    lax.fori_loop(1, _NCH // _NBUF, body, 0)

    cB = _NCH - 1
    wait_gather(cB, cB % _NBUF)
    start_write(cB, cB % _NBUF)
    for c in range(_NCH - _NBUF, _NCH):
        wait_write(c, c % _NBUF)


def kernel(input_ids, weight):
    ids_t = input_ids.astype(jnp.int32).T.reshape(_T)
    table = _prep(weight, jnp.asarray(_EMU))
    out = _gather(table, ids_t)
    return out.reshape(_P, _L, _B, _D).transpose(0, 2, 1, 3)


# speedup vs baseline: 1.0377x; 1.0025x over previous
import functools

import numpy as np
import jax
import jax.numpy as jnp
from jax import lax
from jax.experimental import pallas as pl
from jax.experimental.pallas import tpu as pltpu
from jax.experimental.pallas import tpu_sc as plsc

_P = 8
_V = 1000
_D = 128
_B = 1024
_L = 50
_T = _B * _L
_EPS = 0.01


def _threefry2x32_np(k1, k2, x0, x1):
    def rotl(x, d):
        return ((x << np.uint32(d)) | (x >> np.uint32(32 - d))).astype(np.uint32)

    def rnds(x0, x1, rots):
        for r in rots:
            x0 = (x0 + x1).astype(np.uint32)
            x1 = rotl(x1, r)
            x1 = x1 ^ x0
        return x0, x1

    r0, r1 = (13, 15, 26, 6), (17, 29, 16, 24)
    ks0, ks1 = np.uint32(k1), np.uint32(k2)
    ks2 = np.uint32(ks0 ^ ks1 ^ np.uint32(0x1BD11BDA))
    x0 = (x0 + ks0).astype(np.uint32)
    x1 = (x1 + ks1).astype(np.uint32)
    for i, (ka, kb, rr) in enumerate(
        [(ks1, ks2, r0), (ks2, ks0, r1), (ks0, ks1, r0), (ks1, ks2, r1), (ks2, ks0, r0)]
    ):
        x0, x1 = rnds(x0, x1, rr)
        x0 = (x0 + ka).astype(np.uint32)
        x1 = (x1 + kb + np.uint32(i + 1)).astype(np.uint32)
    return x0, x1


def _emu_np():
    b1, b2 = _threefry2x32_np(
        np.uint32(0), np.uint32(42),
        np.array([0, 0], np.uint32), np.array([0, 1], np.uint32),
    )
    n = _P * _V * _D
    i = np.arange(n, dtype=np.uint64)
    hi = (i >> np.uint64(32)).astype(np.uint32)
    lo = (i & np.uint64(0xFFFFFFFF)).astype(np.uint32)
    bb1, bb2 = _threefry2x32_np(b1[1], b2[1], hi, lo)
    mu = ((bb1 ^ bb2) & np.uint32(1)).astype(np.float32) * 2.0 - 1.0
    return (np.float32(_EPS) * mu).reshape(_P * _V, _D)


_EMU = _emu_np()

def _prep_body(w_ref, emu_ref, tbl_ref):
    tbl_ref[...] = w_ref[...] + emu_ref[...]


def _prep(weight, emu):
    return pl.pallas_call(
        _prep_body,
        grid=(_P,),
        in_specs=[
            pl.BlockSpec((_V, _D), lambda p: (0, 0)),
            pl.BlockSpec((_V, _D), lambda p: (p, 0)),
        ],
        out_specs=pl.BlockSpec((_V, _D), lambda p: (p, 0)),
        out_shape=jax.ShapeDtypeStruct((_P * _V, _D), jnp.float32),
    )(weight, emu)


_info = plsc.get_sparse_core_info()
_NC = _info.num_cores
_NS = _info.num_subcores
_NW = _NC * _NS
_QP = _NW // _P
_RPT = (_P * _T) // _NW
_C = 64
_NCH = _RPT // _C
_NBUF = 4

_mesh = plsc.VectorSubcoreMesh(core_axis_name="c", subcore_axis_name="s")


@functools.partial(
    pl.kernel,
    mesh=_mesh,
    out_type=jax.ShapeDtypeStruct((_P * _T, _D), jnp.float32),
    scratch_types=[
        pltpu.VMEM((_RPT,), jnp.int32),
        pltpu.VMEM((_NBUF, _C, _D), jnp.float32),
        pltpu.VMEM_SHARED((_P * _V, _D), jnp.float32),
        pltpu.SemaphoreType.DMA((_NBUF,)),
        pltpu.SemaphoreType.DMA((_NBUF,)),
        pltpu.SemaphoreType.DMA,
    ],
)
def _gather(tbl_hbm, ids_hbm, out_hbm, ids_v, rows_v, tbl_sh, gsem, wsem, ssem):
    sub = lax.axis_index("s")
    wid = sub * _NC + lax.axis_index("c")
    rbase = wid * _RPT

    @pl.when(sub < _P)
    def _stage():
        v0 = sub * _V
        pltpu.async_copy(tbl_hbm.at[pl.ds(v0, _V)], tbl_sh.at[pl.ds(v0, _V)], ssem)

    pltpu.sync_copy(ids_hbm.at[pl.ds((wid % _QP) * _RPT, _RPT)], ids_v)
    poff = (wid // _QP) * _V

    def addoff(i, carry):
        for jj in range(4):
            sl = pl.ds(i * 64 + jj * 16, 16)
            ids_v[sl] = ids_v[sl] + poff
        return carry

    lax.fori_loop(0, _RPT // 64, addoff, 0)

    @pl.when(sub < _P)
    def _stage_wait():
        v0 = sub * _V
        pltpu.make_async_copy(
            tbl_hbm.at[pl.ds(v0, _V)], tbl_sh.at[pl.ds(v0, _V)], ssem
        ).wait()

    plsc.subcore_barrier()

    def start_gather(c, b):
        return pltpu.async_copy(
            tbl_sh.at[ids_v.at[pl.ds(c * _C, _C)]], rows_v.at[b], gsem.at[b]
        )

    def wait_gather(c, b):
        pltpu.make_async_copy(
            tbl_sh.at[ids_v.at[pl.ds(c * _C, _C)]], rows_v.at[b], gsem.at[b]
        ).wait()

    def start_write(c, b):
        return pltpu.async_copy(
            rows_v.at[b], out_hbm.at[pl.ds(rbase + c * _C, _C)], wsem.at[b]
        )

    def wait_write(c, b):
        pltpu.make_async_copy(
            rows_v.at[b], out_hbm.at[pl.ds(rbase + c * _C, _C)], wsem.at[b]
        ).wait()

    for b in range(_NBUF):
        start_gather(b, b)
    for c in range(_NBUF - 1):
        wait_gather(c, c)
        start_write(c, c)

    def body(i, carry):
        g = i * _NBUF
        for b in range(_NBUF):
            c = g + b
            wait_write(c - _NBUF, b)
            start_gather(c, b)
            b2 = (b - 1) % _NBUF
            wait_gather(c - 1, b2)
            start_write(c - 1, b2)
        return carry

    lax.fori_loop(1, _NCH // _NBUF, body, 0)

    cB = _NCH - 1
    wait_gather(cB, cB % _NBUF)
    start_write(cB, cB % _NBUF)
    for c in range(_NCH - _NBUF, _NCH):
        wait_write(c, c % _NBUF)


def kernel(input_ids, weight):
    ids_t = input_ids.astype(jnp.int32).T.reshape(_T)
    table = _prep(weight, jnp.asarray(_EMU))
    out = _gather(table, ids_t)
    return out.reshape(_P, _L, _B, _D).transpose(0, 2, 1, 3)
